# baseline (speedup 1.0000x reference)
import functools

import jax
import jax.numpy as jnp
from jax.experimental import pallas as pl
from jax.experimental.pallas import tpu as pltpu


def _width_interp_matrix(out_size, in_size):
    if in_size == 1:
        return jnp.ones((1, out_size), dtype=jnp.float32)
    scale = (in_size - 1) / (out_size - 1)
    src = jnp.arange(out_size, dtype=jnp.float32) * scale
    i0 = jnp.clip(jnp.floor(src).astype(jnp.int32), 0, in_size - 1)
    i1 = jnp.clip(i0 + 1, 0, in_size - 1)
    w1 = src - i0.astype(jnp.float32)
    w0 = 1.0 - w1
    oh0 = jax.nn.one_hot(i0, in_size, dtype=jnp.float32)
    oh1 = jax.nn.one_hot(i1, in_size, dtype=jnp.float32)
    return (w0[:, None] * oh0 + w1[:, None] * oh1).T


def _upsample_kernel(h, wxt_ref, x_ref, o_ref):
    x = x_ref[...]
    tb, _, w = x.shape
    wout = wxt_ref.shape[1]
    s = 2 * h - 1

    xd = pltpu.roll(x, 1, 1)
    xu = pltpu.roll(x, h - 1, 1)

    k = jax.lax.broadcasted_iota(jnp.int32, (tb, h, w), 1).astype(jnp.float32)
    inv_s = jnp.float32(1.0 / s)
    a = k * inv_s
    even = a * xd + (1.0 - a) * x
    d = (jnp.float32(h - 1) - k) * inv_s
    odd = (1.0 - d) * x + d * xu

    wxt = wxt_ref[...]
    even2 = jnp.dot(even.reshape(tb * h, w), wxt,
                    preferred_element_type=jnp.float32)
    odd2 = jnp.dot(odd.reshape(tb * h, w), wxt,
                   preferred_element_type=jnp.float32)
    o_ref[:, :, :wout] = even2.reshape(tb, h, wout).astype(o_ref.dtype)
    o_ref[:, :, wout:] = odd2.reshape(tb, h, wout).astype(o_ref.dtype)


@jax.jit
def _up2x(x):
    N, C, H, W = x.shape
    Hout, Wout = 2 * H, 2 * W
    B = N * C
    out_dtype = x.dtype
    in_b = x.dtype.itemsize

    tb = 32
    while B % tb != 0 or B // tb < 2:
        tb //= 2
    g = B // tb

    wxt = _width_interp_matrix(Wout, W)
    x3 = x.reshape(B, H, W)
    cost = pl.CostEstimate(
        flops=2 * B * 2 * H * W * Wout + 8 * B * H * W,
        transcendentals=0,
        bytes_accessed=int(B * H * W * in_b * 5),
    )
    out = pl.pallas_call(
        functools.partial(_upsample_kernel, H),
        out_shape=jax.ShapeDtypeStruct((B, H, 2 * Wout), out_dtype),
        grid=(g,),
        in_specs=[
            pl.BlockSpec((W, Wout), lambda i: (0, 0)),
            pl.BlockSpec((tb, H, W), lambda i: (i, 0, 0)),
        ],
        out_specs=pl.BlockSpec((tb, H, 2 * Wout), lambda i: (i, 0, 0)),
        compiler_params=pltpu.CompilerParams(
            dimension_semantics=("parallel",),
            vmem_limit_bytes=48 * 1024 * 1024),
        cost_estimate=cost,
    )(wxt, x3)
    return out.reshape(N, C, Hout, Wout)


def kernel(x):
    return _up2x(x)


# speedup vs baseline: 2.4825x; 2.4825x over previous
import functools

import jax
import jax.numpy as jnp
from jax.experimental import pallas as pl
from jax.experimental.pallas import tpu as pltpu


def _width_interp_matrix(out_size, in_size):
    if in_size == 1:
        return jnp.ones((1, out_size), dtype=jnp.float32)
    scale = (in_size - 1) / (out_size - 1)
    src = jnp.arange(out_size, dtype=jnp.float32) * scale
    i0 = jnp.clip(jnp.floor(src).astype(jnp.int32), 0, in_size - 1)
    i1 = jnp.clip(i0 + 1, 0, in_size - 1)
    w1 = src - i0.astype(jnp.float32)
    w0 = 1.0 - w1
    oh0 = jax.nn.one_hot(i0, in_size, dtype=jnp.float32)
    oh1 = jax.nn.one_hot(i1, in_size, dtype=jnp.float32)
    return (w0[:, None] * oh0 + w1[:, None] * oh1).T


def _upsample_kernel(h, wxt_ref, x_ref, o_ref):
    x = x_ref[...]
    tb, _, w = x.shape
    wout = wxt_ref.shape[1]
    s = 2 * h - 1

    xd = pltpu.roll(x, 1, 1)
    xu = pltpu.roll(x, h - 1, 1)

    k = jax.lax.broadcasted_iota(jnp.int32, (tb, h, w), 1).astype(jnp.float32)
    inv_s = jnp.float32(1.0 / s)
    a = k * inv_s
    even = a * xd + (1.0 - a) * x
    d = (jnp.float32(h - 1) - k) * inv_s
    odd = (1.0 - d) * x + d * xu

    wxt = wxt_ref[...]
    even2 = jnp.dot(even.reshape(tb * h, w), wxt,
                    preferred_element_type=jnp.float32)
    odd2 = jnp.dot(odd.reshape(tb * h, w), wxt,
                   preferred_element_type=jnp.float32)
    o_ref[:, pl.ds(0, h, 2), :] = even2.reshape(tb, h, wout).astype(o_ref.dtype)
    o_ref[:, pl.ds(1, h, 2), :] = odd2.reshape(tb, h, wout).astype(o_ref.dtype)


@jax.jit
def _up2x(x):
    N, C, H, W = x.shape
    Hout, Wout = 2 * H, 2 * W
    B = N * C
    out_dtype = x.dtype
    in_b = x.dtype.itemsize

    tb = 32
    while B % tb != 0 or B // tb < 2:
        tb //= 2
    g = B // tb

    wxt = _width_interp_matrix(Wout, W)
    x3 = x.reshape(B, H, W)
    cost = pl.CostEstimate(
        flops=2 * B * 2 * H * W * Wout + 8 * B * H * W,
        transcendentals=0,
        bytes_accessed=int(B * H * W * in_b * 5),
    )
    out = pl.pallas_call(
        functools.partial(_upsample_kernel, H),
        out_shape=jax.ShapeDtypeStruct((B, Hout, Wout), out_dtype),
        grid=(g,),
        in_specs=[
            pl.BlockSpec((W, Wout), lambda i: (0, 0)),
            pl.BlockSpec((tb, H, W), lambda i: (i, 0, 0)),
        ],
        out_specs=pl.BlockSpec((tb, Hout, Wout), lambda i: (i, 0, 0)),
        compiler_params=pltpu.CompilerParams(
            dimension_semantics=("parallel",),
            vmem_limit_bytes=48 * 1024 * 1024),
        cost_estimate=cost,
    )(wxt, x3)
    return out.reshape(N, C, Hout, Wout)


def kernel(x):
    return _up2x(x)


# speedup vs baseline: 3.1355x; 1.2630x over previous
import functools

import jax
import jax.numpy as jnp
from jax.experimental import pallas as pl
from jax.experimental.pallas import tpu as pltpu


def _width_interp_matrix(out_size, in_size):
    if in_size == 1:
        return jnp.ones((1, out_size), dtype=jnp.float32)
    scale = (in_size - 1) / (out_size - 1)
    src = jnp.arange(out_size, dtype=jnp.float32) * scale
    i0 = jnp.clip(jnp.floor(src).astype(jnp.int32), 0, in_size - 1)
    i1 = jnp.clip(i0 + 1, 0, in_size - 1)
    w1 = src - i0.astype(jnp.float32)
    w0 = 1.0 - w1
    oh0 = jax.nn.one_hot(i0, in_size, dtype=jnp.float32)
    oh1 = jax.nn.one_hot(i1, in_size, dtype=jnp.float32)
    return (w0[:, None] * oh0 + w1[:, None] * oh1).T


def _upsample_kernel(h, wxt_ref, x_ref, o_ref):
    x = x_ref[...]
    tb, _, w = x.shape
    wout = wxt_ref.shape[1]
    s = 2 * h - 1

    xd = pltpu.roll(x, 1, 1)
    xu = pltpu.roll(x, h - 1, 1)

    k = jax.lax.broadcasted_iota(jnp.int32, (tb, h, w), 1).astype(jnp.float32)
    inv_s = jnp.float32(1.0 / s)
    a = k * inv_s
    even = a * xd + (1.0 - a) * x
    d = (jnp.float32(h - 1) - k) * inv_s
    odd = (1.0 - d) * x + d * xu

    wxt = wxt_ref[...]
    even2 = jnp.dot(even.reshape(tb * h, w), wxt,
                    preferred_element_type=jnp.float32)
    odd2 = jnp.dot(odd.reshape(tb * h, w), wxt,
                   preferred_element_type=jnp.float32)
    o_ref[:, pl.ds(0, h, 2), :] = even2.reshape(tb, h, wout).astype(o_ref.dtype)
    o_ref[:, pl.ds(1, h, 2), :] = odd2.reshape(tb, h, wout).astype(o_ref.dtype)


@jax.jit
def _up2x(x):
    N, C, H, W = x.shape
    Hout, Wout = 2 * H, 2 * W
    B = N * C
    out_dtype = x.dtype
    in_b = x.dtype.itemsize

    tb = 64
    while B % tb != 0 or B // tb < 2:
        tb //= 2
    g = B // tb

    wxt = _width_interp_matrix(Wout, W)
    x3 = x.reshape(B, H, W)
    cost = pl.CostEstimate(
        flops=2 * B * 2 * H * W * Wout + 8 * B * H * W,
        transcendentals=0,
        bytes_accessed=int(B * H * W * in_b * 5),
    )
    out = pl.pallas_call(
        functools.partial(_upsample_kernel, H),
        out_shape=jax.ShapeDtypeStruct((B, Hout, Wout), out_dtype),
        grid=(g,),
        in_specs=[
            pl.BlockSpec((W, Wout), lambda i: (0, 0)),
            pl.BlockSpec((tb, H, W), lambda i: (i, 0, 0)),
        ],
        out_specs=pl.BlockSpec((tb, Hout, Wout), lambda i: (i, 0, 0)),
        compiler_params=pltpu.CompilerParams(
            dimension_semantics=("parallel",),
            vmem_limit_bytes=48 * 1024 * 1024),
        cost_estimate=cost,
    )(wxt, x3)
    return out.reshape(N, C, Hout, Wout)


def kernel(x):
    return _up2x(x)


# speedup vs baseline: 3.3485x; 1.0679x over previous
import functools

import jax
import jax.numpy as jnp
from jax.experimental import pallas as pl
from jax.experimental.pallas import tpu as pltpu


def _width_interp_matrix(out_size, in_size):
    if in_size == 1:
        return jnp.ones((1, out_size), dtype=jnp.float32)
    scale = (in_size - 1) / (out_size - 1)
    src = jnp.arange(out_size, dtype=jnp.float32) * scale
    i0 = jnp.clip(jnp.floor(src).astype(jnp.int32), 0, in_size - 1)
    i1 = jnp.clip(i0 + 1, 0, in_size - 1)
    w1 = src - i0.astype(jnp.float32)
    w0 = 1.0 - w1
    oh0 = jax.nn.one_hot(i0, in_size, dtype=jnp.float32)
    oh1 = jax.nn.one_hot(i1, in_size, dtype=jnp.float32)
    return (w0[:, None] * oh0 + w1[:, None] * oh1).T


def _upsample_kernel(h, wxt_ref, x_ref, o_ref):
    x = x_ref[...]
    tb, _, w = x.shape
    wout = wxt_ref.shape[1]
    s = 2 * h - 1

    xd = pltpu.roll(x, 1, 1)
    xu = pltpu.roll(x, h - 1, 1)

    k = jax.lax.broadcasted_iota(jnp.int32, (tb, h, w), 1).astype(jnp.float32)
    inv_s = jnp.float32(1.0 / s)
    a = k * inv_s
    even = a * xd + (1.0 - a) * x
    d = (jnp.float32(h - 1) - k) * inv_s
    odd = (1.0 - d) * x + d * xu

    wxt = wxt_ref[...]
    even2 = jnp.dot(even.reshape(tb * h, w), wxt,
                    preferred_element_type=jnp.float32)
    odd2 = jnp.dot(odd.reshape(tb * h, w), wxt,
                   preferred_element_type=jnp.float32)
    o_ref[:, pl.ds(0, h, 2), :] = even2.reshape(tb, h, wout).astype(o_ref.dtype)
    o_ref[:, pl.ds(1, h, 2), :] = odd2.reshape(tb, h, wout).astype(o_ref.dtype)


@jax.jit
def _up2x(x):
    N, C, H, W = x.shape
    Hout, Wout = 2 * H, 2 * W
    B = N * C
    out_dtype = x.dtype
    in_b = x.dtype.itemsize

    tb = 128
    while B % tb != 0 or B // tb < 2:
        tb //= 2
    g = B // tb

    wxt = _width_interp_matrix(Wout, W)
    x3 = x.reshape(B, H, W)
    cost = pl.CostEstimate(
        flops=2 * B * 2 * H * W * Wout + 8 * B * H * W,
        transcendentals=0,
        bytes_accessed=int(B * H * W * in_b * 5),
    )
    out = pl.pallas_call(
        functools.partial(_upsample_kernel, H),
        out_shape=jax.ShapeDtypeStruct((B, Hout, Wout), out_dtype),
        grid=(g,),
        in_specs=[
            pl.BlockSpec((W, Wout), lambda i: (0, 0)),
            pl.BlockSpec((tb, H, W), lambda i: (i, 0, 0)),
        ],
        out_specs=pl.BlockSpec((tb, Hout, Wout), lambda i: (i, 0, 0)),
        compiler_params=pltpu.CompilerParams(
            dimension_semantics=("parallel",),
            vmem_limit_bytes=48 * 1024 * 1024),
        cost_estimate=cost,
    )(wxt, x3)
    return out.reshape(N, C, Hout, Wout)


def kernel(x):
    return _up2x(x)


# speedup vs baseline: 3.4096x; 1.0182x over previous
import functools

import jax
import jax.numpy as jnp
from jax.experimental import pallas as pl
from jax.experimental.pallas import tpu as pltpu


def _width_interp_matrix(out_size, in_size):
    if in_size == 1:
        return jnp.ones((1, out_size), dtype=jnp.float32)
    scale = (in_size - 1) / (out_size - 1)
    src = jnp.arange(out_size, dtype=jnp.float32) * scale
    i0 = jnp.clip(jnp.floor(src).astype(jnp.int32), 0, in_size - 1)
    i1 = jnp.clip(i0 + 1, 0, in_size - 1)
    w1 = src - i0.astype(jnp.float32)
    w0 = 1.0 - w1
    oh0 = jax.nn.one_hot(i0, in_size, dtype=jnp.float32)
    oh1 = jax.nn.one_hot(i1, in_size, dtype=jnp.float32)
    return (w0[:, None] * oh0 + w1[:, None] * oh1).T


def _upsample_kernel(h, wxt_ref, x_ref, o_ref):
    x = x_ref[...]
    tb, _, w = x.shape
    wout = wxt_ref.shape[1]
    s = 2 * h - 1

    xd = pltpu.roll(x, 1, 1)
    xu = pltpu.roll(x, h - 1, 1)

    k = jax.lax.broadcasted_iota(jnp.int32, (tb, h, w), 1).astype(jnp.float32)
    inv_s = jnp.float32(1.0 / s)
    a = k * inv_s
    even = a * xd + (1.0 - a) * x
    d = (jnp.float32(h - 1) - k) * inv_s
    odd = (1.0 - d) * x + d * xu

    wxt = wxt_ref[...]
    even2 = jnp.dot(even.reshape(tb * h, w), wxt,
                    preferred_element_type=jnp.float32)
    odd2 = jnp.dot(odd.reshape(tb * h, w), wxt,
                   preferred_element_type=jnp.float32)
    o_ref[:, pl.ds(0, h, 2), :] = even2.reshape(tb, h, wout).astype(o_ref.dtype)
    o_ref[:, pl.ds(1, h, 2), :] = odd2.reshape(tb, h, wout).astype(o_ref.dtype)


@jax.jit
def _up2x(x):
    N, C, H, W = x.shape
    Hout, Wout = 2 * H, 2 * W
    B = N * C
    out_dtype = x.dtype
    in_b = x.dtype.itemsize

    tb = 256
    while B % tb != 0 or B // tb < 2:
        tb //= 2
    g = B // tb

    wxt = _width_interp_matrix(Wout, W)
    x3 = x.reshape(B, H, W)
    cost = pl.CostEstimate(
        flops=2 * B * 2 * H * W * Wout + 8 * B * H * W,
        transcendentals=0,
        bytes_accessed=int(B * H * W * in_b * 5),
    )
    out = pl.pallas_call(
        functools.partial(_upsample_kernel, H),
        out_shape=jax.ShapeDtypeStruct((B, Hout, Wout), out_dtype),
        grid=(g,),
        in_specs=[
            pl.BlockSpec((W, Wout), lambda i: (0, 0)),
            pl.BlockSpec((tb, H, W), lambda i: (i, 0, 0)),
        ],
        out_specs=pl.BlockSpec((tb, Hout, Wout), lambda i: (i, 0, 0)),
        compiler_params=pltpu.CompilerParams(
            dimension_semantics=("parallel",),
            vmem_limit_bytes=56 * 1024 * 1024),
        cost_estimate=cost,
    )(wxt, x3)
    return out.reshape(N, C, Hout, Wout)


def kernel(x):
    return _up2x(x)
